# baseline (speedup 1.0000x reference)
import functools

import jax
import jax.numpy as jnp
from jax import lax
from jax.experimental import pallas as pl
from jax.experimental.pallas import tpu as pltpu
from jax.experimental.pallas import tpu_sc as plsc

VOCAB = 1000000
EMBED_DIM = 64
INPUT_SIZE = 20
TARGET_DIM = 10
BATCH = 16384

N_ROWS = BATCH * INPUT_SIZE


def _make_sc_gather():
    info = plsc.get_sparse_core_info()
    NC, NS = info.num_cores, info.num_subcores
    NW = NC * NS
    rows_per_w = N_ROWS // NW
    CH = 128
    NCH = rows_per_w // CH
    NBUF = 4

    mesh = plsc.VectorSubcoreMesh(core_axis_name="c", subcore_axis_name="s")

    @functools.partial(
        pl.kernel,
        mesh=mesh,
        out_type=jax.ShapeDtypeStruct((N_ROWS, EMBED_DIM), jnp.bfloat16),
        compiler_params=pltpu.CompilerParams(use_tc_tiling_on_sc=False),
        scratch_types=[
            pltpu.VMEM((rows_per_w,), jnp.int32),
            pltpu.VMEM((CH, EMBED_DIM), jnp.bfloat16),
            pltpu.VMEM((CH, EMBED_DIM), jnp.bfloat16),
            pltpu.VMEM((CH, EMBED_DIM), jnp.bfloat16),
            pltpu.VMEM((CH, EMBED_DIM), jnp.bfloat16),
            pltpu.SemaphoreType.DMA,
            pltpu.SemaphoreType.DMA,
            pltpu.SemaphoreType.DMA,
            pltpu.SemaphoreType.DMA,
            pltpu.SemaphoreType.DMA,
            pltpu.SemaphoreType.DMA,
            pltpu.SemaphoreType.DMA,
            pltpu.SemaphoreType.DMA,
        ],
    )
    def gather_k(table_hbm, idx_hbm, out_hbm, idx_v, r0, r1, r2, r3,
                 g0, g1, g2, g3, w0, w1, w2, w3):
        wid = lax.axis_index("s") * NC + lax.axis_index("c")
        base = wid * rows_per_w
        pltpu.sync_copy(idx_hbm.at[pl.ds(base, rows_per_w)], idx_v)

        rows = (r0, r1, r2, r3)
        sg = (g0, g1, g2, g3)
        sw = (w0, w1, w2, w3)

        def fire_gather(i, b):
            pltpu.async_copy(
                table_hbm.at[idx_v.at[pl.ds(i * CH, CH)]],
                rows[b], sg[b])

        def wait_gather(b):
            pltpu.make_async_copy(
                table_hbm.at[idx_v.at[pl.ds(0, CH)]], rows[b], sg[b]).wait()

        def wait_writeback(b):
            pltpu.make_async_copy(
                rows[b], out_hbm.at[pl.ds(base, CH)], sw[b]).wait()

        def outer(p, carry):
            for b in range(NBUF):
                @pl.when(p > 0)
                def _():
                    wait_writeback(b)
                fire_gather(p * NBUF + b, b)
            for b in range(NBUF):
                wait_gather(b)
                pltpu.async_copy(
                    rows[b],
                    out_hbm.at[pl.ds(base + (p * NBUF + b) * CH, CH)],
                    sw[b])
            return carry

        lax.fori_loop(0, NCH // NBUF, outer, 0)
        for b in range(NBUF):
            wait_writeback(b)

    return gather_k


_sc_gather = _make_sc_gather()

_TC_BLK = 256


def _tc_body(f_ref, w_ref, b_ref, o_ref):
    pieces = [
        jnp.maximum(
            f_ref[pl.ds(i * _TC_BLK, _TC_BLK), :].astype(jnp.float32), 0.0)
        for i in range(INPUT_SIZE)
    ]
    f = jnp.concatenate(pieces, axis=1)
    acc = lax.dot_general(
        f, w_ref[...], (((1,), (1,)), ((), ())),
        preferred_element_type=jnp.float32)
    o_ref[...] = acc + b_ref[...]


def _tc_linear(features, W, b2):
    grid = (BATCH // _TC_BLK,)
    return pl.pallas_call(
        _tc_body,
        grid=grid,
        in_specs=[
            pl.BlockSpec((_TC_BLK * INPUT_SIZE, EMBED_DIM), lambda i: (i, 0)),
            pl.BlockSpec((TARGET_DIM, INPUT_SIZE * EMBED_DIM), lambda i: (0, 0)),
            pl.BlockSpec((1, TARGET_DIM), lambda i: (0, 0)),
        ],
        out_specs=pl.BlockSpec((_TC_BLK, TARGET_DIM), lambda i: (i, 0)),
        out_shape=jax.ShapeDtypeStruct((BATCH, TARGET_DIM), jnp.float32),
    )(features, W, b2)


def kernel(x, embedding, W, b):
    nblk = BATCH // _TC_BLK
    idx = (x.astype(jnp.int32)
           .reshape(nblk, _TC_BLK, INPUT_SIZE)
           .transpose(0, 2, 1)
           .reshape(-1))
    table = embedding.astype(jnp.bfloat16)
    feats = _sc_gather(table, idx)
    return _tc_linear(feats, W, b.reshape(1, TARGET_DIM))


# speedup vs baseline: 1.5316x; 1.5316x over previous
import functools

import jax
import jax.numpy as jnp
from jax import lax
from jax.experimental import pallas as pl
from jax.experimental.pallas import tpu as pltpu
from jax.experimental.pallas import tpu_sc as plsc

VOCAB = 1000000
EMBED_DIM = 64
PAD_DIM = 128
INPUT_SIZE = 20
TARGET_DIM = 10
BATCH = 16384

N_ROWS = BATCH * INPUT_SIZE


def _make_sc_gather():
    info = plsc.get_sparse_core_info()
    NC, NS = info.num_cores, info.num_subcores
    NW = NC * NS
    rows_per_w = N_ROWS // NW
    CH = 128
    NCH = rows_per_w // CH
    NBUF = 4

    mesh = plsc.VectorSubcoreMesh(core_axis_name="c", subcore_axis_name="s")

    @functools.partial(
        pl.kernel,
        mesh=mesh,
        out_type=jax.ShapeDtypeStruct((N_ROWS, PAD_DIM), jnp.float32),
        scratch_types=[
            pltpu.VMEM((rows_per_w,), jnp.int32),
            pltpu.VMEM((CH, PAD_DIM), jnp.float32),
            pltpu.VMEM((CH, PAD_DIM), jnp.float32),
            pltpu.VMEM((CH, PAD_DIM), jnp.float32),
            pltpu.VMEM((CH, PAD_DIM), jnp.float32),
            pltpu.SemaphoreType.DMA,
            pltpu.SemaphoreType.DMA,
            pltpu.SemaphoreType.DMA,
            pltpu.SemaphoreType.DMA,
            pltpu.SemaphoreType.DMA,
            pltpu.SemaphoreType.DMA,
            pltpu.SemaphoreType.DMA,
            pltpu.SemaphoreType.DMA,
        ],
    )
    def gather_k(table_hbm, idx_hbm, out_hbm, idx_v, r0, r1, r2, r3,
                 g0, g1, g2, g3, w0, w1, w2, w3):
        wid = lax.axis_index("s") * NC + lax.axis_index("c")
        base = wid * rows_per_w
        pltpu.sync_copy(idx_hbm.at[pl.ds(base, rows_per_w)], idx_v)

        rows = (r0, r1, r2, r3)
        sg = (g0, g1, g2, g3)
        sw = (w0, w1, w2, w3)

        def fire_gather(i, b):
            pltpu.async_copy(
                table_hbm.at[idx_v.at[pl.ds(i * CH, CH)]],
                rows[b], sg[b])

        def wait_gather(b):
            pltpu.make_async_copy(
                table_hbm.at[idx_v.at[pl.ds(0, CH)]], rows[b], sg[b]).wait()

        def wait_writeback(b):
            pltpu.make_async_copy(
                rows[b], out_hbm.at[pl.ds(base, CH)], sw[b]).wait()

        def outer(p, carry):
            for b in range(NBUF):
                @pl.when(p > 0)
                def _():
                    wait_writeback(b)
                fire_gather(p * NBUF + b, b)
            for b in range(NBUF):
                wait_gather(b)
                pltpu.async_copy(
                    rows[b],
                    out_hbm.at[pl.ds(base + (p * NBUF + b) * CH, CH)],
                    sw[b])
            return carry

        lax.fori_loop(0, NCH // NBUF, outer, 0)
        for b in range(NBUF):
            wait_writeback(b)

    return gather_k


_sc_gather = _make_sc_gather()

_TC_BLK = 256


def _tc_body(f_ref, w_ref, b_ref, o_ref):
    pieces = [
        jnp.maximum(f_ref[pl.ds(i * _TC_BLK, _TC_BLK), pl.ds(0, EMBED_DIM)],
                    0.0)
        for i in range(INPUT_SIZE)
    ]
    f = jnp.concatenate(pieces, axis=1)
    acc = lax.dot_general(
        f, w_ref[...], (((1,), (1,)), ((), ())),
        preferred_element_type=jnp.float32)
    o_ref[...] = acc + b_ref[...]


def _tc_linear(features, W, b2):
    grid = (BATCH // _TC_BLK,)
    return pl.pallas_call(
        _tc_body,
        grid=grid,
        in_specs=[
            pl.BlockSpec((_TC_BLK * INPUT_SIZE, PAD_DIM), lambda i: (i, 0)),
            pl.BlockSpec((TARGET_DIM, INPUT_SIZE * EMBED_DIM), lambda i: (0, 0)),
            pl.BlockSpec((1, TARGET_DIM), lambda i: (0, 0)),
        ],
        out_specs=pl.BlockSpec((_TC_BLK, TARGET_DIM), lambda i: (i, 0)),
        out_shape=jax.ShapeDtypeStruct((BATCH, TARGET_DIM), jnp.float32),
    )(features, W, b2)


def kernel(x, embedding, W, b):
    nblk = BATCH // _TC_BLK
    idx = (x.astype(jnp.int32)
           .reshape(nblk, _TC_BLK, INPUT_SIZE)
           .transpose(0, 2, 1)
           .reshape(-1))
    table = jnp.pad(embedding, ((0, 0), (0, PAD_DIM - EMBED_DIM)))
    feats = _sc_gather(table, idx)
    return _tc_linear(feats, W, b.reshape(1, TARGET_DIM))


# speedup vs baseline: 2.0847x; 1.3611x over previous
import functools

import jax
import jax.numpy as jnp
from jax import lax
from jax.experimental import pallas as pl
from jax.experimental.pallas import tpu as pltpu
from jax.experimental.pallas import tpu_sc as plsc

VOCAB = 1000000
EMBED_DIM = 64
INPUT_SIZE = 20
TARGET_DIM = 10
BATCH = 16384

N_ROWS = BATCH * INPUT_SIZE


def _make_sc_gather(n_rows):
    info = plsc.get_sparse_core_info()
    NC, NS = info.num_cores, info.num_subcores
    NW = NC * NS
    rows_per_w = n_rows // NW
    CH = 128
    NCH = rows_per_w // CH
    UNROLL = 16

    mesh = plsc.VectorSubcoreMesh(core_axis_name="c", subcore_axis_name="s")

    @functools.partial(
        pl.kernel,
        mesh=mesh,
        out_type=jax.ShapeDtypeStruct((n_rows, EMBED_DIM), jnp.float32),
        scratch_types=[
            pltpu.VMEM((rows_per_w,), jnp.int32),
            pltpu.VMEM((CH, EMBED_DIM), jnp.float32),
            pltpu.VMEM((CH, EMBED_DIM), jnp.float32),
            pltpu.SemaphoreType.DMA,
            pltpu.SemaphoreType.DMA,
            pltpu.SemaphoreType.DMA,
            pltpu.SemaphoreType.DMA,
        ],
    )
    def gather_k(table_hbm, idx_hbm, out_hbm, idx_v, rows0, rows1,
                 sg0, sg1, sw0, sw1):
        wid = lax.axis_index("s") * NC + lax.axis_index("c")
        base = wid * rows_per_w
        pltpu.sync_copy(idx_hbm.at[pl.ds(base, rows_per_w)], idx_v)

        rows = (rows0, rows1)
        sg = (sg0, sg1)
        sw = (sw0, sw1)

        def fire_chunk(i, s):
            def body(g, carry):
                j = g * UNROLL
                vec = idx_v[pl.ds(i * CH + j, UNROLL)]
                for k in range(UNROLL):
                    pltpu.async_copy(
                        table_hbm.at[pl.ds(vec[k], 1)],
                        rows[s].at[pl.ds(j + k, 1)],
                        sg[s])
                return carry

            lax.fori_loop(0, CH // UNROLL, body, 0)

        def drain_gather(s):
            pltpu.make_async_copy(
                table_hbm.at[pl.ds(0, CH)], rows[s], sg[s]).wait()

        def wait_writeback(s):
            pltpu.make_async_copy(
                rows[s], out_hbm.at[pl.ds(base, CH)], sw[s]).wait()

        def outer(p, carry):
            i0 = p * 2
            for b in range(2):
                @pl.when((i0 + b) >= 2)
                def _():
                    wait_writeback(b)
                fire_chunk(i0 + b, b)
            for b in range(2):
                drain_gather(b)
                pltpu.async_copy(
                    rows[b],
                    out_hbm.at[pl.ds(base + (i0 + b) * CH, CH)],
                    sw[b])
            return carry

        lax.fori_loop(0, NCH // 2, outer, 0)
        wait_writeback(0)
        wait_writeback(1)

    return gather_k


_HALF_ROWS = N_ROWS // 2
_sc_gather = _make_sc_gather(_HALF_ROWS)

_TC_BLK = 256


def _tc_body(f_ref, w_ref, b_ref, o_ref):
    pieces = [
        jnp.maximum(f_ref[pl.ds(i * _TC_BLK, _TC_BLK), :], 0.0)
        for i in range(INPUT_SIZE)
    ]
    f = jnp.concatenate(pieces, axis=1)
    acc = lax.dot_general(
        f, w_ref[...], (((1,), (1,)), ((), ())),
        preferred_element_type=jnp.float32)
    o_ref[...] = acc + b_ref[...]


def _tc_linear(features, W, b2, batch):
    grid = (batch // _TC_BLK,)
    return pl.pallas_call(
        _tc_body,
        grid=grid,
        in_specs=[
            pl.BlockSpec((_TC_BLK * INPUT_SIZE, EMBED_DIM), lambda i: (i, 0)),
            pl.BlockSpec((TARGET_DIM, INPUT_SIZE * EMBED_DIM), lambda i: (0, 0)),
            pl.BlockSpec((1, TARGET_DIM), lambda i: (0, 0)),
        ],
        out_specs=pl.BlockSpec((_TC_BLK, TARGET_DIM), lambda i: (i, 0)),
        out_shape=jax.ShapeDtypeStruct((batch, TARGET_DIM), jnp.float32),
    )(features, W, b2)


def kernel(x, embedding, W, b):
    nblk = BATCH // _TC_BLK
    idx = (x.astype(jnp.int32)
           .reshape(nblk, _TC_BLK, INPUT_SIZE)
           .transpose(0, 2, 1)
           .reshape(-1))
    b2 = b.reshape(1, TARGET_DIM)
    outs = []
    for h in range(2):
        idx_h = lax.dynamic_slice_in_dim(idx, h * _HALF_ROWS, _HALF_ROWS)
        feats = _sc_gather(embedding, idx_h)
        outs.append(_tc_linear(feats, W, b2, BATCH // 2))
    return jnp.concatenate(outs, axis=0)
